# baseline (speedup 1.0000x reference)
import jax
import jax.numpy as jnp
from jax import lax
from jax.experimental import pallas as pl
from jax.experimental.pallas import tpu as pltpu
from jax.experimental.pallas import tpu_sc as plsc

NC, NS = 2, 16
NW = NC * NS
RSTAGE = 8
VOCAB = 5
L_SEQ = 200
EMB = 128
UNROLL = 8


def _tc_body(cell_emb_ref, table_ref, w_ref, b_ref, cell_ref, comb_ref):
    cell = lax.dot_general(
        cell_emb_ref[...], w_ref[...],
        dimension_numbers=(((1,), (1,)), ((), ())),
        preferred_element_type=jnp.float32,
    ) + b_ref[...]
    cell_ref[...] = cell
    comb_ref[...] = table_ref[:VOCAB][None, :, :] + cell[:, None, :]


def _sc_body(comb_hbm, seqs_hbm, out_hbm,
             seq_v, comb_v, out0, out1, s0, s1):
    wid = lax.axis_index("s") * NC + lax.axis_index("c")
    rows_per_w = seqs_hbm.shape[0] // (NW * L_SEQ)
    row0 = wid * rows_per_w
    n_chunks = rows_per_w // RSTAGE

    outs = (out0, out1)
    sems = (s0, s1)

    iota16 = lax.iota(jnp.int32, 16)
    n_grp = (L_SEQ + 15) // 16

    def expand_row(seq_row, comb_base, outbuf):
        def grp(g, carry):
            del carry
            tok0 = jnp.minimum(g * 16, L_SEQ - 16)
            sv = seq_v[pl.ds(seq_row * L_SEQ + tok0, 16)]
            src = (sv + comb_base) * EMB
            dst = (iota16 + tok0) * EMB
            for _ in range(EMB):
                x = plsc.load_gather(comb_v, [src])
                plsc.store_scatter(outbuf, [dst], x)
                src = src + 1
                dst = dst + 1
            return 0
        lax.fori_loop(0, n_grp, grp, 0)

    def chunk(ci, carry):
        del carry
        rbase = row0 + ci * RSTAGE
        pltpu.sync_copy(seqs_hbm.at[pl.ds(rbase * L_SEQ, RSTAGE * L_SEQ)],
                        seq_v)
        pltpu.sync_copy(
            comb_hbm.at[pl.ds(rbase * VOCAB * EMB, RSTAGE * VOCAB * EMB)],
            comb_v)

        def pair(m, carry2):
            del carry2
            for p in range(2):
                r = 2 * m + p
                @pl.when(jnp.logical_or(ci > 0, m > 0))
                def _(p=p):
                    pltpu.make_async_copy(
                        outs[p], out_hbm.at[pl.ds(0, L_SEQ * EMB)],
                        sems[p]).wait()
                expand_row(r, r * VOCAB, outs[p])
                pltpu.async_copy(
                    outs[p],
                    out_hbm.at[pl.ds((rbase + r) * L_SEQ * EMB, L_SEQ * EMB)],
                    sems[p])
            return 0

        lax.fori_loop(0, RSTAGE // 2, pair, 0)
        return 0

    lax.fori_loop(0, n_chunks, chunk, 0)

    for p in range(2):
        pltpu.make_async_copy(
            outs[p], out_hbm.at[pl.ds(0, L_SEQ * EMB)], sems[p]).wait()


def kernel(seqs, cell_emb, table, W, b):
    B, L = seqs.shape
    vocab, emb = table.shape
    cin = cell_emb.shape[1]

    vpad = 8
    table_p = jnp.zeros((vpad, emb), jnp.float32).at[:vocab].set(table)
    b2 = b.reshape(1, emb)

    BBLK = 512
    cell, comb = pl.pallas_call(
        _tc_body,
        grid=(B // BBLK,),
        in_specs=[
            pl.BlockSpec((BBLK, cin), lambda i: (i, 0)),
            pl.BlockSpec((vpad, emb), lambda i: (0, 0)),
            pl.BlockSpec((emb, cin), lambda i: (0, 0)),
            pl.BlockSpec((1, emb), lambda i: (0, 0)),
        ],
        out_specs=[
            pl.BlockSpec((BBLK, emb), lambda i: (i, 0)),
            pl.BlockSpec((BBLK, vocab, emb), lambda i: (i, 0, 0)),
        ],
        out_shape=[
            jax.ShapeDtypeStruct((B, emb), jnp.float32),
            jax.ShapeDtypeStruct((B, vocab, emb), jnp.float32),
        ],
    )(cell_emb, table_p, W, b2)

    comb_flat = comb.reshape(B * vocab * emb)

    mesh = plsc.VectorSubcoreMesh(core_axis_name="c", subcore_axis_name="s")
    total_flat = pl.kernel(
        _sc_body,
        out_type=jax.ShapeDtypeStruct((B * L * emb,), jnp.float32),
        mesh=mesh,
        compiler_params=pltpu.CompilerParams(needs_layout_passes=False),
        scratch_types=[
            pltpu.VMEM((RSTAGE * L,), jnp.int32),
            pltpu.VMEM((RSTAGE * VOCAB * emb,), jnp.float32),
            pltpu.VMEM((L * emb,), jnp.float32),
            pltpu.VMEM((L * emb,), jnp.float32),
            pltpu.SemaphoreType.DMA,
            pltpu.SemaphoreType.DMA,
        ],
    )(comb_flat, seqs.reshape(B * L))

    return (total_flat.reshape(B, L, emb), cell)


# speedup vs baseline: 6.4611x; 6.4611x over previous
import jax
import jax.numpy as jnp
from jax import lax
from jax.experimental import pallas as pl
from jax.experimental.pallas import tpu as pltpu
from jax.experimental.pallas import tpu_sc as plsc

NC, NS = 2, 16
NW = NC * NS
ROWS_PER_XFER = 128
NSLOT = 6
LAG = 5


def _tc_body(seqs_ref, cell_emb_ref, table_ref, w_ref, b_ref,
             cell_ref, comb_ref, idx_ref):
    bblk, L = seqs_ref.shape
    i = pl.program_id(0)

    cell = lax.dot_general(
        cell_emb_ref[...], w_ref[...],
        dimension_numbers=(((1,), (1,)), ((), ())),
        preferred_element_type=jnp.float32,
    ) + b_ref[...]
    cell_ref[...] = cell

    vocab = comb_ref.shape[1]
    comb_ref[...] = table_ref[:vocab][None, :, :] + cell[:, None, :]

    row = i * bblk + lax.broadcasted_iota(jnp.int32, (bblk, L), 0)
    idx_ref[...] = vocab * row + seqs_ref[...]


def _sc_body(comb_hbm, idx_hbm, out_hbm, idx_v, *bufsems):
    bufs = bufsems[:NSLOT]
    gsems = bufsems[NSLOT:2 * NSLOT]
    ssems = bufsems[2 * NSLOT:]
    wid = lax.axis_index("s") * NC + lax.axis_index("c")
    n_xfer = idx_hbm.shape[1]
    base = wid * n_xfer * ROWS_PER_XFER

    pltpu.sync_copy(idx_hbm.at[wid], idx_v)

    def scatter_wait(p, sem):
        pltpu.make_async_copy(
            bufs[p], out_hbm.at[pl.ds(base, ROWS_PER_XFER)], sem).wait()

    def round_(jj, carry):
        del carry
        j0 = jj * NSLOT
        for p in range(NSLOT):
            j = j0 + p
            @pl.when(j < n_xfer)
            def _(p=p, j=j):
                @pl.when(j >= NSLOT)
                def _():
                    scatter_wait(p, ssems[p])

                pltpu.async_copy(comb_hbm.at[idx_v.at[j]], bufs[p], gsems[p])

            t = j - LAG
            q = (p - LAG) % NSLOT

            @pl.when(jnp.logical_and(t >= 0, t < n_xfer))
            def _(q=q, t=t):
                pltpu.make_async_copy(
                    comb_hbm.at[idx_v.at[0]], bufs[q], gsems[q]).wait()
                pltpu.async_copy(
                    bufs[q],
                    out_hbm.at[pl.ds(base + t * ROWS_PER_XFER,
                                     ROWS_PER_XFER)],
                    ssems[q])

        return 0

    n_rounds = (n_xfer + LAG + NSLOT - 1) // NSLOT
    lax.fori_loop(0, n_rounds, round_, 0)

    for p in range(NSLOT):
        scatter_wait(p, ssems[p])


def kernel(seqs, cell_emb, table, W, b):
    B, L = seqs.shape
    vocab, emb = table.shape
    cin = cell_emb.shape[1]

    vpad = 8
    table_p = jnp.zeros((vpad, emb), jnp.float32).at[:vocab].set(table)
    b2 = b.reshape(1, emb)

    BBLK = 512
    cell, comb, idx = pl.pallas_call(
        _tc_body,
        grid=(B // BBLK,),
        in_specs=[
            pl.BlockSpec((BBLK, L), lambda i: (i, 0)),
            pl.BlockSpec((BBLK, cin), lambda i: (i, 0)),
            pl.BlockSpec((vpad, emb), lambda i: (0, 0)),
            pl.BlockSpec((emb, cin), lambda i: (0, 0)),
            pl.BlockSpec((1, emb), lambda i: (0, 0)),
        ],
        out_specs=[
            pl.BlockSpec((BBLK, emb), lambda i: (i, 0)),
            pl.BlockSpec((BBLK, vocab, emb), lambda i: (i, 0, 0)),
            pl.BlockSpec((BBLK, L), lambda i: (i, 0)),
        ],
        out_shape=[
            jax.ShapeDtypeStruct((B, emb), jnp.float32),
            jax.ShapeDtypeStruct((B, vocab, emb), jnp.float32),
            jax.ShapeDtypeStruct((B, L), jnp.int32),
        ],
    )(seqs, cell_emb, table_p, W, b2)

    comb_flat = comb.reshape(B * vocab, emb)
    tokens = B * L
    n_xfer = tokens // (NW * ROWS_PER_XFER)
    idx3 = idx.reshape(NW, n_xfer, ROWS_PER_XFER)

    mesh = plsc.VectorSubcoreMesh(core_axis_name="c", subcore_axis_name="s")
    total_flat = pl.kernel(
        _sc_body,
        out_type=jax.ShapeDtypeStruct((tokens, emb), jnp.float32),
        mesh=mesh,
        scratch_types=(
            [pltpu.VMEM((n_xfer, ROWS_PER_XFER), jnp.int32)]
            + [pltpu.VMEM((ROWS_PER_XFER, emb), jnp.float32)] * NSLOT
            + [pltpu.SemaphoreType.DMA] * (2 * NSLOT)
        ),
    )(comb_flat, idx3)

    return (total_flat.reshape(B, L, emb), cell)
